# baseline (speedup 1.0000x reference)
import functools
import math

import jax
import jax.numpy as jnp
from jax.experimental import pallas as pl
from jax.experimental.pallas import tpu as pltpu

B, S, D = 1, 2048, 2048
H, KVH, HD = 16, 4, 128
E, K, F = 8, 2, 768
EPS = 1e-06
THETA = 10000.0

BT = 256
CB = 512
NHB = CB // HD


def _rope(x, tb):
    half = HD // 2
    j = jax.lax.broadcasted_iota(jnp.int32, (BT, half), 1).astype(jnp.float32)
    inv = jnp.exp(j * (-math.log(THETA) / half))
    pos = (tb * BT + jax.lax.broadcasted_iota(jnp.int32, (BT, half), 0)
           ).astype(jnp.float32)
    f = pos * inv
    cos = jnp.cos(f)
    sin = jnp.sin(f)
    x1 = x[:, :half]
    x2 = x[:, half:]
    return jnp.concatenate([x1 * cos - x2 * sin, x2 * cos + x1 * sin], axis=-1)


def _qkv_kernel(x_ref, w_ref, ln_ref, qn_ref, kn_ref, o_ref):
    tb = pl.program_id(0)
    c = pl.program_id(1)
    x = x_ref[...]
    ms = jnp.mean(x * x, axis=-1, keepdims=True)
    h = x * jax.lax.rsqrt(ms + EPS) * ln_ref[...]
    raw = jnp.dot(h, w_ref[...], preferred_element_type=jnp.float32)
    nw = jnp.where(c < 4, qn_ref[...], kn_ref[...])
    segs = []
    for j in range(NHB):
        seg = raw[:, j * HD:(j + 1) * HD]
        m = jnp.mean(seg * seg, axis=-1, keepdims=True)
        seg = seg * jax.lax.rsqrt(m + EPS) * nw
        segs.append(_rope(seg, tb))
    o_ref[...] = jnp.where(c == 5, raw, jnp.concatenate(segs, axis=-1))


def _attn_kernel(q_ref, k_ref, v_ref, o_ref):
    qb = pl.program_id(1)
    q = q_ref[...]
    scale = 1.0 / math.sqrt(HD)
    rowg = qb * BT + jax.lax.broadcasted_iota(jnp.int32, (BT, BT), 0)
    coll = jax.lax.broadcasted_iota(jnp.int32, (BT, BT), 1)

    def body(kb, carry):
        m, l, acc = carry
        kc = k_ref[pl.ds(kb * BT, BT), :]
        vc = v_ref[pl.ds(kb * BT, BT), :]
        s = jax.lax.dot_general(q, kc, (((1,), (1,)), ((), ())),
                                preferred_element_type=jnp.float32) * scale
        s = jnp.where(kb * BT + coll <= rowg, s, -1e30)
        mnew = jnp.maximum(m, jnp.max(s, axis=-1, keepdims=True))
        p = jnp.exp(s - mnew)
        alpha = jnp.exp(m - mnew)
        l = l * alpha + jnp.sum(p, axis=-1, keepdims=True)
        acc = acc * alpha + jnp.dot(p, vc, preferred_element_type=jnp.float32)
        return mnew, l, acc

    m0 = jnp.full((BT, 1), -1e30, jnp.float32)
    l0 = jnp.zeros((BT, 1), jnp.float32)
    a0 = jnp.zeros((BT, HD), jnp.float32)
    m, l, acc = jax.lax.fori_loop(0, qb + 1, body, (m0, l0, a0))
    o_ref[...] = acc / l


def _oproj_router_kernel(o_ref, w_ref, res_ref, ln_ref, rw_ref,
                         x_ref, h_ref, idx_ref, wt_ref):
    x1 = res_ref[...] + jnp.dot(o_ref[...], w_ref[...],
                                preferred_element_type=jnp.float32)
    x_ref[...] = x1
    ms = jnp.mean(x1 * x1, axis=-1, keepdims=True)
    h = x1 * jax.lax.rsqrt(ms + EPS) * ln_ref[...]
    h_ref[...] = h
    logits = jnp.dot(h, rw_ref[...], preferred_element_type=jnp.float32)
    m = jnp.max(logits, axis=-1, keepdims=True)
    p = jnp.exp(logits - m)
    probs = p / jnp.sum(p, axis=-1, keepdims=True)
    idx = jax.lax.broadcasted_iota(jnp.int32, (BT, E), 1)
    m1 = jnp.max(probs, axis=-1, keepdims=True)
    i1 = jnp.min(jnp.where(probs == m1, idx, E), axis=-1, keepdims=True)
    probs2 = jnp.where(idx == i1, -1.0, probs)
    m2 = jnp.max(probs2, axis=-1, keepdims=True)
    i2 = jnp.min(jnp.where(probs2 == m2, idx, E), axis=-1, keepdims=True)
    tot = m1 + m2
    idx_ref[...] = jnp.concatenate([i1, i2], axis=1)
    wt_ref[...] = jnp.concatenate([m1 / tot, m2 / tot], axis=1)


def _moe_group_kernel(te_ref, hg_ref, wg_ref, wu_ref, wd_ref, w_ref, yg_ref):
    h = hg_ref[...]
    g = jnp.dot(h, wg_ref[0], preferred_element_type=jnp.float32)
    u = jnp.dot(h, wu_ref[0], preferred_element_type=jnp.float32)
    a = (g / (1.0 + jnp.exp(-g))) * u
    y = jnp.dot(a, wd_ref[0], preferred_element_type=jnp.float32)
    yg_ref[...] = y * w_ref[...][:, 0:1]


def kernel(hidden_states, ln1_w, Wq, Wk, Wv, q_norm_w, k_norm_w, Wo, ln2_w,
           router_W, W_gate, W_up, W_down):
    x = hidden_states.reshape(S, D)
    nt = S // BT

    Wqkv = jnp.concatenate([Wq, Wk, Wv], axis=1)
    QKVW = Wqkv.shape[1]
    qkv = pl.pallas_call(
        _qkv_kernel,
        grid=(nt, QKVW // CB),
        in_specs=[
            pl.BlockSpec((BT, D), lambda t, c: (t, 0)),
            pl.BlockSpec((D, CB), lambda t, c: (0, c)),
            pl.BlockSpec((1, D), lambda t, c: (0, 0)),
            pl.BlockSpec((1, HD), lambda t, c: (0, 0)),
            pl.BlockSpec((1, HD), lambda t, c: (0, 0)),
        ],
        out_specs=pl.BlockSpec((BT, CB), lambda t, c: (t, c)),
        out_shape=jax.ShapeDtypeStruct((S, QKVW), jnp.float32),
    )(x, Wqkv, ln1_w.reshape(1, D), q_norm_w.reshape(1, HD),
      k_norm_w.reshape(1, HD))
    q = qkv[:, :H * HD]
    k = qkv[:, H * HD:H * HD + KVH * HD]
    v = qkv[:, H * HD + KVH * HD:]

    rep = H // KVH
    o = pl.pallas_call(
        _attn_kernel,
        grid=(H, nt),
        in_specs=[
            pl.BlockSpec((BT, HD), lambda h, t: (t, h)),
            pl.BlockSpec((S, HD), lambda h, t: (0, h // rep)),
            pl.BlockSpec((S, HD), lambda h, t: (0, h // rep)),
        ],
        out_specs=pl.BlockSpec((BT, HD), lambda h, t: (t, h)),
        out_shape=jax.ShapeDtypeStruct((S, H * HD), jnp.float32),
    )(q, k, v)

    x1, h2, idx12, w12 = pl.pallas_call(
        _oproj_router_kernel,
        grid=(nt,),
        in_specs=[
            pl.BlockSpec((BT, H * HD), lambda t: (t, 0)),
            pl.BlockSpec((H * HD, D), lambda t: (0, 0)),
            pl.BlockSpec((BT, D), lambda t: (t, 0)),
            pl.BlockSpec((1, D), lambda t: (0, 0)),
            pl.BlockSpec((D, E), lambda t: (0, 0)),
        ],
        out_specs=[
            pl.BlockSpec((BT, D), lambda t: (t, 0)),
            pl.BlockSpec((BT, D), lambda t: (t, 0)),
            pl.BlockSpec((BT, K), lambda t: (t, 0)),
            pl.BlockSpec((BT, K), lambda t: (t, 0)),
        ],
        out_shape=[
            jax.ShapeDtypeStruct((S, D), jnp.float32),
            jax.ShapeDtypeStruct((S, D), jnp.float32),
            jax.ShapeDtypeStruct((S, K), jnp.int32),
            jax.ShapeDtypeStruct((S, K), jnp.float32),
        ],
    )(o, Wo, x, ln2_w.reshape(1, D), router_W)

    NTOT = S * K
    RB = 128
    NP = NTOT + E * RB
    NT2 = NP // RB
    e_flat = idx12.reshape(NTOT)
    w_flat = w12.reshape(NTOT)
    t_flat = jnp.arange(NTOT, dtype=jnp.int32) // K
    oh = (e_flat[:, None] == jnp.arange(E, dtype=jnp.int32)[None, :]
          ).astype(jnp.int32)
    rank_all = jnp.cumsum(oh, axis=0) - oh
    rank = jnp.sum(rank_all * oh, axis=1)
    counts = jnp.sum(oh, axis=0)
    pg = ((counts + RB - 1) // RB) * RB
    poff = jnp.concatenate([jnp.zeros((1,), jnp.int32), jnp.cumsum(pg)[:-1]])
    pos = poff[e_flat] + rank
    src_tok = jnp.zeros((NP,), jnp.int32).at[pos].set(t_flat)
    wpad = jnp.zeros((NP,), jnp.float32).at[pos].set(w_flat)
    cumpad = poff + pg
    tstart = jnp.arange(NT2, dtype=jnp.int32) * RB
    tile_expert = jnp.minimum(
        jnp.sum((cumpad[None, :] <= tstart[:, None]).astype(jnp.int32), axis=1),
        E - 1)
    w2d = jnp.broadcast_to(wpad[:, None], (NP, 8))

    hg = jnp.take(h2, src_tok, axis=0)

    yg = pl.pallas_call(
        _moe_group_kernel,
        grid_spec=pltpu.PrefetchScalarGridSpec(
            num_scalar_prefetch=1,
            grid=(NT2,),
            in_specs=[
                pl.BlockSpec((RB, D), lambda t, te: (t, 0)),
                pl.BlockSpec((1, D, F), lambda t, te: (te[t], 0, 0)),
                pl.BlockSpec((1, D, F), lambda t, te: (te[t], 0, 0)),
                pl.BlockSpec((1, F, D), lambda t, te: (te[t], 0, 0)),
                pl.BlockSpec((RB, 8), lambda t, te: (t, 0)),
            ],
            out_specs=pl.BlockSpec((RB, D), lambda t, te: (t, 0)),
        ),
        out_shape=jax.ShapeDtypeStruct((NP, D), jnp.float32),
    )(tile_expert, hg, W_gate, W_up, W_down, w2d)

    out = x1.at[src_tok].add(yg)
    return out.reshape(B, S, D)


# speedup vs baseline: 1.0913x; 1.0913x over previous
import functools
import math

import jax
import jax.numpy as jnp
from jax.experimental import pallas as pl
from jax.experimental.pallas import tpu as pltpu

B, S, D = 1, 2048, 2048
H, KVH, HD = 16, 4, 128
E, K, F = 8, 2, 768
EPS = 1e-06
THETA = 10000.0

BT = 256
CB = 512
NHB = CB // HD


def _rope(x, tb):
    half = HD // 2
    j = jax.lax.broadcasted_iota(jnp.int32, (BT, half), 1).astype(jnp.float32)
    inv = jnp.exp(j * (-math.log(THETA) / half))
    pos = (tb * BT + jax.lax.broadcasted_iota(jnp.int32, (BT, half), 0)
           ).astype(jnp.float32)
    f = pos * inv
    cos = jnp.cos(f)
    sin = jnp.sin(f)
    x1 = x[:, :half]
    x2 = x[:, half:]
    return jnp.concatenate([x1 * cos - x2 * sin, x2 * cos + x1 * sin], axis=-1)


def _qkv_kernel(x_ref, w_ref, ln_ref, qn_ref, kn_ref, o_ref):
    tb = pl.program_id(0)
    c = pl.program_id(1)
    x = x_ref[...]
    ms = jnp.mean(x * x, axis=-1, keepdims=True)
    h = x * jax.lax.rsqrt(ms + EPS) * ln_ref[...]
    raw = jnp.dot(h, w_ref[...], preferred_element_type=jnp.float32)
    nw = jnp.where(c < 4, qn_ref[...], kn_ref[...])
    segs = []
    for j in range(NHB):
        seg = raw[:, j * HD:(j + 1) * HD]
        m = jnp.mean(seg * seg, axis=-1, keepdims=True)
        seg = seg * jax.lax.rsqrt(m + EPS) * nw
        segs.append(_rope(seg, tb))
    o_ref[...] = jnp.where(c == 5, raw, jnp.concatenate(segs, axis=-1))


def _attn_kernel(q_ref, k_ref, v_ref, o_ref):
    qb = pl.program_id(1)
    q = q_ref[...]
    k = k_ref[...]
    s = jax.lax.dot_general(q, k, (((1,), (1,)), ((), ())),
                            preferred_element_type=jnp.float32)
    s = s * (1.0 / math.sqrt(HD))
    row = qb * BT + jax.lax.broadcasted_iota(jnp.int32, (BT, S), 0)
    col = jax.lax.broadcasted_iota(jnp.int32, (BT, S), 1)
    s = jnp.where(col <= row, s, -1e30)
    m = jnp.max(s, axis=-1, keepdims=True)
    p = jnp.exp(s - m)
    l = jnp.sum(p, axis=-1, keepdims=True)
    o = jnp.dot(p, v_ref[...], preferred_element_type=jnp.float32)
    o_ref[...] = o / l


def _oproj_router_kernel(o_ref, w_ref, res_ref, ln_ref, rw_ref,
                         x_ref, h_ref, idx_ref, wt_ref):
    x1 = res_ref[...] + jnp.dot(o_ref[...], w_ref[...],
                                preferred_element_type=jnp.float32)
    x_ref[...] = x1
    ms = jnp.mean(x1 * x1, axis=-1, keepdims=True)
    h = x1 * jax.lax.rsqrt(ms + EPS) * ln_ref[...]
    h_ref[...] = h
    logits = jnp.dot(h, rw_ref[...], preferred_element_type=jnp.float32)
    m = jnp.max(logits, axis=-1, keepdims=True)
    p = jnp.exp(logits - m)
    probs = p / jnp.sum(p, axis=-1, keepdims=True)
    idx = jax.lax.broadcasted_iota(jnp.int32, (BT, E), 1)
    m1 = jnp.max(probs, axis=-1, keepdims=True)
    i1 = jnp.min(jnp.where(probs == m1, idx, E), axis=-1, keepdims=True)
    probs2 = jnp.where(idx == i1, -1.0, probs)
    m2 = jnp.max(probs2, axis=-1, keepdims=True)
    i2 = jnp.min(jnp.where(probs2 == m2, idx, E), axis=-1, keepdims=True)
    tot = m1 + m2
    idx_ref[...] = jnp.concatenate([i1, i2], axis=1)
    wt_ref[...] = jnp.concatenate([m1 / tot, m2 / tot], axis=1)


def _moe_group_kernel(te_ref, hg_ref, wg_ref, wu_ref, wd_ref, w_ref, yg_ref):
    h = hg_ref[...]
    g = jnp.dot(h, wg_ref[0], preferred_element_type=jnp.float32)
    u = jnp.dot(h, wu_ref[0], preferred_element_type=jnp.float32)
    a = (g / (1.0 + jnp.exp(-g))) * u
    y = jnp.dot(a, wd_ref[0], preferred_element_type=jnp.float32)
    yg_ref[...] = y * w_ref[...][:, 0:1]


def kernel(hidden_states, ln1_w, Wq, Wk, Wv, q_norm_w, k_norm_w, Wo, ln2_w,
           router_W, W_gate, W_up, W_down):
    x = hidden_states.reshape(S, D)
    nt = S // BT

    Wqkv = jnp.concatenate([Wq, Wk, Wv], axis=1)
    QKVW = Wqkv.shape[1]
    qkv = pl.pallas_call(
        _qkv_kernel,
        grid=(nt, QKVW // CB),
        in_specs=[
            pl.BlockSpec((BT, D), lambda t, c: (t, 0)),
            pl.BlockSpec((D, CB), lambda t, c: (0, c)),
            pl.BlockSpec((1, D), lambda t, c: (0, 0)),
            pl.BlockSpec((1, HD), lambda t, c: (0, 0)),
            pl.BlockSpec((1, HD), lambda t, c: (0, 0)),
        ],
        out_specs=pl.BlockSpec((BT, CB), lambda t, c: (t, c)),
        out_shape=jax.ShapeDtypeStruct((S, QKVW), jnp.float32),
    )(x, Wqkv, ln1_w.reshape(1, D), q_norm_w.reshape(1, HD),
      k_norm_w.reshape(1, HD))
    q = qkv[:, :H * HD]
    k = qkv[:, H * HD:H * HD + KVH * HD]
    v = qkv[:, H * HD + KVH * HD:]

    rep = H // KVH
    o = pl.pallas_call(
        _attn_kernel,
        grid=(H, nt),
        in_specs=[
            pl.BlockSpec((BT, HD), lambda h, t: (t, h)),
            pl.BlockSpec((S, HD), lambda h, t: (0, h // rep)),
            pl.BlockSpec((S, HD), lambda h, t: (0, h // rep)),
        ],
        out_specs=pl.BlockSpec((BT, HD), lambda h, t: (t, h)),
        out_shape=jax.ShapeDtypeStruct((S, H * HD), jnp.float32),
    )(q, k, v)

    x1, h2, idx12, w12 = pl.pallas_call(
        _oproj_router_kernel,
        grid=(nt,),
        in_specs=[
            pl.BlockSpec((BT, H * HD), lambda t: (t, 0)),
            pl.BlockSpec((H * HD, D), lambda t: (0, 0)),
            pl.BlockSpec((BT, D), lambda t: (t, 0)),
            pl.BlockSpec((1, D), lambda t: (0, 0)),
            pl.BlockSpec((D, E), lambda t: (0, 0)),
        ],
        out_specs=[
            pl.BlockSpec((BT, D), lambda t: (t, 0)),
            pl.BlockSpec((BT, D), lambda t: (t, 0)),
            pl.BlockSpec((BT, K), lambda t: (t, 0)),
            pl.BlockSpec((BT, K), lambda t: (t, 0)),
        ],
        out_shape=[
            jax.ShapeDtypeStruct((S, D), jnp.float32),
            jax.ShapeDtypeStruct((S, D), jnp.float32),
            jax.ShapeDtypeStruct((S, K), jnp.int32),
            jax.ShapeDtypeStruct((S, K), jnp.float32),
        ],
    )(o, Wo, x, ln2_w.reshape(1, D), router_W)

    NTOT = S * K
    RB = 128
    NP = NTOT + E * RB
    NT2 = NP // RB
    e_flat = idx12.reshape(NTOT)
    w_flat = w12.reshape(NTOT)
    t_flat = jnp.arange(NTOT, dtype=jnp.int32) // K
    oh = (e_flat[:, None] == jnp.arange(E, dtype=jnp.int32)[None, :]
          ).astype(jnp.int32)
    rank_all = jnp.cumsum(oh, axis=0) - oh
    rank = jnp.sum(rank_all * oh, axis=1)
    counts = jnp.sum(oh, axis=0)
    pg = ((counts + RB - 1) // RB) * RB
    poff = jnp.concatenate([jnp.zeros((1,), jnp.int32), jnp.cumsum(pg)[:-1]])
    pos = poff[e_flat] + rank
    src_tok = jnp.zeros((NP,), jnp.int32).at[pos].set(t_flat)
    wpad = jnp.zeros((NP,), jnp.float32).at[pos].set(w_flat)
    cumpad = poff + pg
    tstart = jnp.arange(NT2, dtype=jnp.int32) * RB
    tile_expert = jnp.minimum(
        jnp.sum((cumpad[None, :] <= tstart[:, None]).astype(jnp.int32), axis=1),
        E - 1)
    w2d = jnp.broadcast_to(wpad[:, None], (NP, 8))

    hg = jnp.take(h2, src_tok, axis=0)

    yg = pl.pallas_call(
        _moe_group_kernel,
        grid_spec=pltpu.PrefetchScalarGridSpec(
            num_scalar_prefetch=1,
            grid=(NT2,),
            in_specs=[
                pl.BlockSpec((RB, D), lambda t, te: (t, 0)),
                pl.BlockSpec((1, D, F), lambda t, te: (te[t], 0, 0)),
                pl.BlockSpec((1, D, F), lambda t, te: (te[t], 0, 0)),
                pl.BlockSpec((1, F, D), lambda t, te: (te[t], 0, 0)),
                pl.BlockSpec((RB, 8), lambda t, te: (t, 0)),
            ],
            out_specs=pl.BlockSpec((RB, D), lambda t, te: (t, 0)),
        ),
        out_shape=jax.ShapeDtypeStruct((NP, D), jnp.float32),
    )(tile_expert, hg, W_gate, W_up, W_down, w2d)

    out = x1.at[src_tok].add(yg)
    return out.reshape(B, S, D)


# speedup vs baseline: 1.2909x; 1.1829x over previous
import functools
import math

import jax
import jax.numpy as jnp
from jax.experimental import pallas as pl
from jax.experimental.pallas import tpu as pltpu

B, S, D = 1, 2048, 2048
H, KVH, HD = 16, 4, 128
E, K, F = 8, 2, 768
EPS = 1e-06
THETA = 10000.0

BT = 256
CB = 512
NHB = CB // HD


def _rope(x, cos, sin):
    half = HD // 2
    x1 = x[:, :half]
    x2 = x[:, half:]
    return jnp.concatenate([x1 * cos - x2 * sin, x2 * cos + x1 * sin], axis=-1)


def _qkv_kernel(x_ref, w_ref, ln_ref, qn_ref, kn_ref, cos_ref, sin_ref, o_ref):
    c = pl.program_id(0)
    x = x_ref[...]
    ms = jnp.mean(x * x, axis=-1, keepdims=True)
    h = x * jax.lax.rsqrt(ms + EPS) * ln_ref[...]
    raw = jnp.dot(h, w_ref[...], preferred_element_type=jnp.float32)
    nw = jnp.where(c < 4, qn_ref[...], kn_ref[...])
    cos = cos_ref[...]
    sin = sin_ref[...]
    segs = []
    for j in range(NHB):
        seg = raw[:, j * HD:(j + 1) * HD]
        m = jnp.mean(seg * seg, axis=-1, keepdims=True)
        seg = seg * jax.lax.rsqrt(m + EPS) * nw
        segs.append(_rope(seg, cos, sin))
    o_ref[...] = jnp.where(c == 5, raw, jnp.concatenate(segs, axis=-1))


def _attn_kernel(q_ref, k_ref, v_ref, o_ref):
    qb = pl.program_id(1)
    q = q_ref[...] * (1.0 / math.sqrt(HD))
    k = k_ref[...]
    s = jax.lax.dot_general(q, k, (((1,), (1,)), ((), ())),
                            preferred_element_type=jnp.float32)
    row = qb * BT + jax.lax.broadcasted_iota(jnp.int32, (BT, S), 0)
    col = jax.lax.broadcasted_iota(jnp.int32, (BT, S), 1)
    p = jnp.exp(jnp.where(col <= row, s, -1e30))
    l = jnp.sum(p, axis=-1, keepdims=True)
    o = jnp.dot(p, v_ref[...], preferred_element_type=jnp.float32)
    o_ref[...] = o / l


def _oproj_router_kernel(o_ref, w_ref, res_ref, ln_ref, rw_ref,
                         x_ref, h_ref, idx_ref, wt_ref):
    x1 = res_ref[...] + jnp.dot(o_ref[...], w_ref[...],
                                preferred_element_type=jnp.float32)
    x_ref[...] = x1
    ms = jnp.mean(x1 * x1, axis=-1, keepdims=True)
    h = x1 * jax.lax.rsqrt(ms + EPS) * ln_ref[...]
    h_ref[...] = h
    logits = jnp.dot(h, rw_ref[...], preferred_element_type=jnp.float32)
    m = jnp.max(logits, axis=-1, keepdims=True)
    p = jnp.exp(logits - m)
    probs = p / jnp.sum(p, axis=-1, keepdims=True)
    idx = jax.lax.broadcasted_iota(jnp.int32, (BT, E), 1)
    m1 = jnp.max(probs, axis=-1, keepdims=True)
    i1 = jnp.min(jnp.where(probs == m1, idx, E), axis=-1, keepdims=True)
    probs2 = jnp.where(idx == i1, -1.0, probs)
    m2 = jnp.max(probs2, axis=-1, keepdims=True)
    i2 = jnp.min(jnp.where(probs2 == m2, idx, E), axis=-1, keepdims=True)
    tot = m1 + m2
    idx_ref[...] = jnp.concatenate([i1, i2], axis=1)
    wt_ref[...] = jnp.concatenate([m1 / tot, m2 / tot], axis=1)


def _moe_group_kernel(te_ref, hg_ref, wg_ref, wu_ref, wd_ref, w_ref, yg_ref):
    h = hg_ref[...]
    g = jnp.dot(h, wg_ref[0], preferred_element_type=jnp.float32)
    u = jnp.dot(h, wu_ref[0], preferred_element_type=jnp.float32)
    a = (g / (1.0 + jnp.exp(-g))) * u
    y = jnp.dot(a, wd_ref[0], preferred_element_type=jnp.float32)
    yg_ref[...] = y * w_ref[...][:, 0:1]


def kernel(hidden_states, ln1_w, Wq, Wk, Wv, q_norm_w, k_norm_w, Wo, ln2_w,
           router_W, W_gate, W_up, W_down):
    x = hidden_states.reshape(S, D)
    nt = S // BT

    Wqkv = jnp.concatenate([Wq, Wk, Wv], axis=1)
    QKVW = Wqkv.shape[1]
    half = HD // 2
    posf = jnp.arange(S, dtype=jnp.float32)
    invf = jnp.exp(jnp.arange(half, dtype=jnp.float32) * (-math.log(THETA) / half))
    ang = posf[:, None] * invf[None, :]
    cos_tab = jnp.cos(ang)
    sin_tab = jnp.sin(ang)
    qkv = pl.pallas_call(
        _qkv_kernel,
        grid=(QKVW // CB, nt),
        in_specs=[
            pl.BlockSpec((BT, D), lambda c, t: (t, 0)),
            pl.BlockSpec((D, CB), lambda c, t: (0, c)),
            pl.BlockSpec((1, D), lambda c, t: (0, 0)),
            pl.BlockSpec((1, HD), lambda c, t: (0, 0)),
            pl.BlockSpec((1, HD), lambda c, t: (0, 0)),
            pl.BlockSpec((BT, half), lambda c, t: (t, 0)),
            pl.BlockSpec((BT, half), lambda c, t: (t, 0)),
        ],
        out_specs=pl.BlockSpec((BT, CB), lambda c, t: (t, c)),
        out_shape=jax.ShapeDtypeStruct((S, QKVW), jnp.float32),
    )(x, Wqkv, ln1_w.reshape(1, D), q_norm_w.reshape(1, HD),
      k_norm_w.reshape(1, HD), cos_tab, sin_tab)
    q = qkv[:, :H * HD]
    k = qkv[:, H * HD:H * HD + KVH * HD]
    v = qkv[:, H * HD + KVH * HD:]

    rep = H // KVH
    o = pl.pallas_call(
        _attn_kernel,
        grid=(H, nt),
        in_specs=[
            pl.BlockSpec((BT, HD), lambda h, t: (t, h)),
            pl.BlockSpec((S, HD), lambda h, t: (0, h // rep)),
            pl.BlockSpec((S, HD), lambda h, t: (0, h // rep)),
        ],
        out_specs=pl.BlockSpec((BT, HD), lambda h, t: (t, h)),
        out_shape=jax.ShapeDtypeStruct((S, H * HD), jnp.float32),
    )(q, k, v)

    x1, h2, idx12, w12 = pl.pallas_call(
        _oproj_router_kernel,
        grid=(nt,),
        in_specs=[
            pl.BlockSpec((BT, H * HD), lambda t: (t, 0)),
            pl.BlockSpec((H * HD, D), lambda t: (0, 0)),
            pl.BlockSpec((BT, D), lambda t: (t, 0)),
            pl.BlockSpec((1, D), lambda t: (0, 0)),
            pl.BlockSpec((D, E), lambda t: (0, 0)),
        ],
        out_specs=[
            pl.BlockSpec((BT, D), lambda t: (t, 0)),
            pl.BlockSpec((BT, D), lambda t: (t, 0)),
            pl.BlockSpec((BT, K), lambda t: (t, 0)),
            pl.BlockSpec((BT, K), lambda t: (t, 0)),
        ],
        out_shape=[
            jax.ShapeDtypeStruct((S, D), jnp.float32),
            jax.ShapeDtypeStruct((S, D), jnp.float32),
            jax.ShapeDtypeStruct((S, K), jnp.int32),
            jax.ShapeDtypeStruct((S, K), jnp.float32),
        ],
    )(o, Wo, x, ln2_w.reshape(1, D), router_W)

    NTOT = S * K
    RB = 128
    NP = NTOT + E * RB
    NT2 = NP // RB
    e_flat = idx12.reshape(NTOT)
    w_flat = w12.reshape(NTOT)
    t_flat = jnp.arange(NTOT, dtype=jnp.int32) // K
    oh = (e_flat[:, None] == jnp.arange(E, dtype=jnp.int32)[None, :]
          ).astype(jnp.int32)
    rank_all = jnp.cumsum(oh, axis=0) - oh
    rank = jnp.sum(rank_all * oh, axis=1)
    counts = jnp.sum(oh, axis=0)
    pg = ((counts + RB - 1) // RB) * RB
    poff = jnp.concatenate([jnp.zeros((1,), jnp.int32), jnp.cumsum(pg)[:-1]])
    pos = poff[e_flat] + rank
    src_tok = jnp.zeros((NP,), jnp.int32).at[pos].set(t_flat)
    wpad = jnp.zeros((NP,), jnp.float32).at[pos].set(w_flat)
    cumpad = poff + pg
    tstart = jnp.arange(NT2, dtype=jnp.int32) * RB
    tile_expert = jnp.minimum(
        jnp.sum((cumpad[None, :] <= tstart[:, None]).astype(jnp.int32), axis=1),
        E - 1)
    w2d = jnp.broadcast_to(wpad[:, None], (NP, 8))

    hg = jnp.take(h2, src_tok, axis=0)

    yg = pl.pallas_call(
        _moe_group_kernel,
        grid_spec=pltpu.PrefetchScalarGridSpec(
            num_scalar_prefetch=1,
            grid=(NT2,),
            in_specs=[
                pl.BlockSpec((RB, D), lambda t, te: (t, 0)),
                pl.BlockSpec((1, D, F), lambda t, te: (te[t], 0, 0)),
                pl.BlockSpec((1, D, F), lambda t, te: (te[t], 0, 0)),
                pl.BlockSpec((1, F, D), lambda t, te: (te[t], 0, 0)),
                pl.BlockSpec((RB, 8), lambda t, te: (t, 0)),
            ],
            out_specs=pl.BlockSpec((RB, D), lambda t, te: (t, 0)),
        ),
        out_shape=jax.ShapeDtypeStruct((NP, D), jnp.float32),
    )(tile_expert, hg, W_gate, W_up, W_down, w2d)

    out = x1.at[src_tok].add(yg)
    return out.reshape(B, S, D)
